# Initial kernel scaffold; baseline (speedup 1.0000x reference)
import jax
import jax.numpy as jnp
from jax.experimental import pallas as pl


def kernel(tokens, table):
    raise NotImplementedError("write your pallas kernel here")


# speedup vs baseline: 1.5637x; 1.5637x over previous
import functools

import jax
import jax.numpy as jnp
from jax import lax
from jax.experimental import pallas as pl
from jax.experimental.pallas import tpu as pltpu
from jax.experimental.pallas import tpu_sc as plsc

_VOCAB = 1000000
_PAD = _VOCAB - 1
_L = 16


@functools.cache
def _make_sc_embed(n_rows, emb, chunk):
    info = plsc.get_sparse_core_info()
    nc, ns = info.num_cores, info.num_subcores
    nw = nc * ns
    rows_per_w = n_rows // nw
    nchunk = rows_per_w // chunk
    mesh = plsc.VectorSubcoreMesh(core_axis_name="c", subcore_axis_name="s")

    @functools.partial(
        pl.kernel,
        mesh=mesh,
        out_type=jax.ShapeDtypeStruct((n_rows, emb), jnp.float32),
        scratch_types=[
            pltpu.VMEM((chunk,), jnp.int32),
            pltpu.VMEM((chunk, emb), jnp.float32),
            pltpu.SemaphoreType.DMA,
        ],
        compiler_params=pltpu.CompilerParams(use_tc_tiling_on_sc=False),
    )
    def k(tokens_hbm, table_hbm, out_hbm, idx_v, rows_v, sem):
        wid = lax.axis_index("s") * nc + lax.axis_index("c")
        wbase = wid * rows_per_w

        def chunk_body(ci, carry):
            base = wbase + ci * chunk
            pltpu.sync_copy(tokens_hbm.at[pl.ds(base, chunk)], idx_v)
            cp = pltpu.async_copy(table_hbm.at[idx_v], rows_v, sem)

            def scan_body(g, acc):
                v = idx_v[pl.ds(g * _L, _L)]
                return acc | jnp.where(v == _PAD, 1, 0)

            acc = lax.fori_loop(0, chunk // _L, scan_body,
                                jnp.zeros((_L,), jnp.int32))
            for sh in (8, 4, 2, 1):
                perm = lax.iota(jnp.int32, _L) ^ sh
                acc = acc | acc.at[perm].get(mode="promise_in_bounds")
            npad = acc[0]
            cp.wait()

            @pl.when(npad > 0)
            def _fix_pads():
                zeros = jnp.zeros((_L,), jnp.float32)

                def fix_group(g, c2):
                    v = idx_v[pl.ds(g * _L, _L)]
                    gacc = jnp.where(v == _PAD, 1, 0)
                    for sh in (8, 4, 2, 1):
                        perm = lax.iota(jnp.int32, _L) ^ sh
                        gacc = gacc | gacc.at[perm].get(
                            mode="promise_in_bounds")

                    @pl.when(gacc[0] > 0)
                    def _():
                        for j in range(_L):
                            @pl.when(v[j] == _PAD)
                            def _zero_row(j=j):
                                r = g * _L + j
                                for h in range(emb // _L):
                                    rows_v[r, pl.ds(h * _L, _L)] = zeros

                    return c2

                lax.fori_loop(0, chunk // _L, fix_group, 0)

            pltpu.sync_copy(rows_v, out_hbm.at[pl.ds(base, chunk)])
            return carry

        lax.fori_loop(0, nchunk, chunk_body, 0)

    return k


@jax.jit
def kernel(tokens, table):
    flat = tokens.reshape(-1)
    out = _make_sc_embed(flat.shape[0], table.shape[1], 2560)(flat, table)
    return out.reshape(*tokens.shape, table.shape[1])
